# baseline (speedup 1.0000x reference)
import functools

import jax
import jax.numpy as jnp
from jax import lax
from jax.experimental import pallas as pl
from jax.experimental.pallas import tpu as pltpu
from jax.experimental.pallas import tpu_sc as plsc

N = 10000
E = 320000
D = 128

NC = 2
NS = 16
NW = NC * NS
EPW = E // NW
CHUNK = 40
NCHUNK = EPW // CHUNK
NPAD = 10240
RPS = NPAD // NS
ZROWS = 32
CW = 16
DA = D + CW


def _make_agg(width):
  mesh = plsc.VectorSubcoreMesh(core_axis_name="c", subcore_axis_name="s")
  out_type = [jax.ShapeDtypeStruct((NC, NPAD, width), jnp.float32)]
  scratch = [
      pltpu.VMEM((NCHUNK, CHUNK), jnp.int32),
      pltpu.VMEM((NCHUNK, CHUNK), jnp.int32),
      pltpu.VMEM((CHUNK, width), jnp.float32),
      pltpu.VMEM((CHUNK, width), jnp.float32),
      pltpu.VMEM((ZROWS, width), jnp.float32),
      pltpu.VMEM_SHARED((NPAD, width), jnp.float32),
      pltpu.SemaphoreType.DMA,
      pltpu.SemaphoreType.DMA,
  ]

  @functools.partial(pl.kernel, mesh=mesh, out_type=out_type,
                     scratch_types=scratch,
                     compiler_params=pltpu.CompilerParams(
                         use_tc_tiling_on_sc=False))
  def agg(x_hbm, src_hbm, dst_hbm, out_hbm,
          sidx, didx, rows0, rows1, zbuf, acc_sh, sem0, sem1):
    c = lax.axis_index("c")
    s = lax.axis_index("s")
    wid = s * NC + c

    pltpu.sync_copy(src_hbm.at[wid], sidx)
    pltpu.sync_copy(dst_hbm.at[wid], didx)

    zv = jnp.zeros((16,), jnp.float32)

    def zb_body(i, carry):
      for j in range(width // 16):
        zbuf[i, pl.ds(j * 16, 16)] = zv
      return carry
    lax.fori_loop(0, ZROWS, zb_body, 0)

    for j in range(RPS // ZROWS):
      pltpu.sync_copy(zbuf, acc_sh.at[pl.ds(s * RPS + j * ZROWS, ZROWS)])
    plsc.subcore_barrier()

    bufs = ((rows0, sem0), (rows1, sem1))
    for b, (rb, sb) in enumerate(bufs):
      pltpu.async_copy(x_hbm.at[sidx.at[b]], rb, sb)

    def step(k, carry):
      for b, (rb, sb) in enumerate(bufs):
        i = 2 * k + b
        pltpu.make_async_copy(x_hbm.at[sidx.at[0]], rb, sb).wait()
        pltpu.sync_copy(rb, acc_sh.at[didx.at[i]], add=True)

        @pl.when(i + 2 < NCHUNK)
        def _():
          pltpu.async_copy(x_hbm.at[sidx.at[i + 2]], rb, sb)
      return carry
    lax.fori_loop(0, NCHUNK // 2, step, 0)

    plsc.subcore_barrier()

    pltpu.sync_copy(acc_sh.at[pl.ds(s * RPS, RPS)],
                    out_hbm.at[c, pl.ds(s * RPS, RPS)])

  return agg


_agg_cnt = _make_agg(DA)
_agg = _make_agg(D)

BLK = 1000


def _tc_body(sum_ref, cnt_ref, x_ref, wl_ref, bl_ref, wr_ref, o_ref, *, relu):
  s = sum_ref[0] + sum_ref[1]
  cnt = cnt_ref[...]
  c = cnt[0, :, D:D + 1] + cnt[1, :, D:D + 1]
  mean = s[:, :D] / jnp.maximum(c, 1.0)
  y = jnp.dot(mean, wl_ref[...], preferred_element_type=jnp.float32)
  y = y + jnp.dot(x_ref[...], wr_ref[...], preferred_element_type=jnp.float32)
  y = y + bl_ref[...]
  if relu:
    y = jnp.maximum(y, 0.0)
  o_ref[...] = y


def _tc_layer(relu, width):
  return pl.pallas_call(
      functools.partial(_tc_body, relu=relu),
      grid=(N // BLK,),
      in_specs=[
          pl.BlockSpec((NC, BLK, width), lambda i: (0, i, 0)),
          pl.BlockSpec((NC, BLK, DA), lambda i: (0, i, 0)),
          pl.BlockSpec((BLK, D), lambda i: (i, 0)),
          pl.BlockSpec((D, D), lambda i: (0, 0)),
          pl.BlockSpec((1, D), lambda i: (0, 0)),
          pl.BlockSpec((D, D), lambda i: (0, 0)),
      ],
      out_specs=pl.BlockSpec((BLK, D), lambda i: (i, 0)),
      out_shape=jax.ShapeDtypeStruct((N, D), jnp.float32),
  )


_tc1 = _tc_layer(True, DA)
_tc2 = _tc_layer(False, D)


def kernel(x, edge_index, W1l, b1l, W1r, W2l, b2l, W2r):
  src = edge_index[0].astype(jnp.int32).reshape(NW, NCHUNK, CHUNK)
  dst = edge_index[1].astype(jnp.int32).reshape(NW, NCHUNK, CHUNK)
  xaug = jnp.concatenate([x, jnp.ones((N, CW), jnp.float32)], axis=1)
  (sum1,) = _agg_cnt(xaug, src, dst)
  h = _tc1(sum1, sum1, x, W1l, b1l.reshape(1, D), W1r)
  (sum2,) = _agg(h, src, dst)
  out = _tc2(sum2, sum1, h, W2l, b2l.reshape(1, D), W2r)
  return out


# speedup vs baseline: 1.0632x; 1.0632x over previous
import functools

import jax
import jax.numpy as jnp
from jax import lax
from jax.experimental import pallas as pl
from jax.experimental.pallas import tpu as pltpu
from jax.experimental.pallas import tpu_sc as plsc

N = 10000
E = 320000
D = 128

NC = 2
NS = 16
NW = NC * NS
EPW = E // NW
CHUNK = 40
NCHUNK = EPW // CHUNK
NB = 2
NPAD = 10240
RPS = NPAD // NS
ZROWS = 16
CW = 16


def _make_agg(with_counts):
  mesh = plsc.VectorSubcoreMesh(core_axis_name="c", subcore_axis_name="s")
  out_type = [jax.ShapeDtypeStruct((NC, NPAD, D), jnp.float32)]
  if with_counts:
    out_type.append(jax.ShapeDtypeStruct((NC, NPAD, CW), jnp.float32))
  scratch = [
      pltpu.VMEM((NCHUNK, CHUNK), jnp.int32),
      pltpu.VMEM((NCHUNK, CHUNK), jnp.int32),
      pltpu.VMEM((CHUNK, D), jnp.float32),
      pltpu.VMEM((CHUNK, D), jnp.float32),
      pltpu.VMEM((ZROWS, D), jnp.float32),
      pltpu.VMEM_SHARED((NPAD, D), jnp.float32),
      pltpu.SemaphoreType.DMA,
      pltpu.SemaphoreType.DMA,
  ]
  if with_counts:
    scratch += [
        pltpu.VMEM((ZROWS, CW), jnp.float32),
        pltpu.VMEM((CHUNK, CW), jnp.float32),
        pltpu.VMEM_SHARED((NPAD, CW), jnp.float32),
    ]

  @functools.partial(pl.kernel, mesh=mesh, out_type=out_type,
                     scratch_types=scratch,
                     compiler_params=pltpu.CompilerParams(
                         use_tc_tiling_on_sc=False))
  def agg(x_hbm, src_hbm, dst_hbm, *refs):
    if with_counts:
      out_hbm, cnt_hbm = refs[0], refs[1]
      (sidx, didx, rows0, rows1, zbuf, acc_sh,
       gs0, gs1, zcnt, ones, cnt_sh) = refs[2:]
    else:
      out_hbm = refs[0]
      (sidx, didx, rows0, rows1, zbuf, acc_sh, gs0, gs1) = refs[1:]
      cnt_hbm = zcnt = ones = cnt_sh = None

    c = lax.axis_index("c")
    s = lax.axis_index("s")
    wid = s * NC + c

    pltpu.sync_copy(src_hbm.at[wid], sidx)
    pltpu.sync_copy(dst_hbm.at[wid], didx)

    zv = jnp.zeros((16,), jnp.float32)
    ov = jnp.ones((16,), jnp.float32)

    def zb_body(i, carry):
      for j in range(D // 16):
        zbuf[i, pl.ds(j * 16, 16)] = zv
      return carry
    lax.fori_loop(0, ZROWS, zb_body, 0)

    if with_counts:
      def zc_body(i, carry):
        zcnt[i, pl.ds(0, 16)] = zv
        return carry
      lax.fori_loop(0, ZROWS, zc_body, 0)

      def on_body(i, carry):
        ones[i, pl.ds(0, 16)] = ov
        return carry
      lax.fori_loop(0, CHUNK, on_body, 0)

    for j in range(RPS // ZROWS):
      pltpu.sync_copy(zbuf, acc_sh.at[pl.ds(s * RPS + j * ZROWS, ZROWS)])
    if with_counts:
      for j in range(RPS // ZROWS):
        pltpu.sync_copy(zcnt, cnt_sh.at[pl.ds(s * RPS + j * ZROWS, ZROWS)])
    plsc.subcore_barrier()

    bufs = ((rows0, gs0), (rows1, gs1))

    def wait_gather(rb, gsem):
      pltpu.make_async_copy(x_hbm.at[sidx.at[0]], rb, gsem).wait()

    def step(i, b, do_gather):
      rb, gsem = bufs[b]
      wait_gather(rb, gsem)
      pltpu.sync_copy(rb, acc_sh.at[didx.at[i]], add=True)
      if with_counts:
        pltpu.sync_copy(ones, cnt_sh.at[didx.at[i]], add=True)
      if do_gather:
        pltpu.async_copy(x_hbm.at[sidx.at[i + 2]], rb, gsem)

    pltpu.async_copy(x_hbm.at[sidx.at[0]], rows0, gs0)
    pltpu.async_copy(x_hbm.at[sidx.at[1]], rows1, gs1)

    def steady(k, carry):
      i = 2 * k
      step(i, 0, True)
      step(i + 1, 1, True)
      return carry
    lax.fori_loop(0, NCHUNK // 2 - 1, steady, 0)

    step(NCHUNK - 2, 0, False)
    step(NCHUNK - 1, 1, False)

    plsc.subcore_barrier()

    pltpu.sync_copy(acc_sh.at[pl.ds(s * RPS, RPS)],
                    out_hbm.at[c, pl.ds(s * RPS, RPS)])
    if with_counts:
      pltpu.sync_copy(cnt_sh.at[pl.ds(s * RPS, RPS)],
                      cnt_hbm.at[c, pl.ds(s * RPS, RPS)])

  return agg


_agg_cnt = _make_agg(True)
_agg = _make_agg(False)

BLK = 1000


def _tc_body(sum_ref, cnt_ref, x_ref, wl_ref, bl_ref, wr_ref, o_ref, *, relu):
  s = sum_ref[0] + sum_ref[1]
  cnt = cnt_ref[...]
  c = cnt[0, :, 0:1] + cnt[1, :, 0:1]
  mean = s / jnp.maximum(c, 1.0)
  y = jnp.dot(mean, wl_ref[...], preferred_element_type=jnp.float32)
  y = y + jnp.dot(x_ref[...], wr_ref[...], preferred_element_type=jnp.float32)
  y = y + bl_ref[...]
  if relu:
    y = jnp.maximum(y, 0.0)
  o_ref[...] = y


def _tc_layer(relu):
  return pl.pallas_call(
      functools.partial(_tc_body, relu=relu),
      grid=(N // BLK,),
      in_specs=[
          pl.BlockSpec((NC, BLK, D), lambda i: (0, i, 0)),
          pl.BlockSpec((NC, BLK, CW), lambda i: (0, i, 0)),
          pl.BlockSpec((BLK, D), lambda i: (i, 0)),
          pl.BlockSpec((D, D), lambda i: (0, 0)),
          pl.BlockSpec((1, D), lambda i: (0, 0)),
          pl.BlockSpec((D, D), lambda i: (0, 0)),
      ],
      out_specs=pl.BlockSpec((BLK, D), lambda i: (i, 0)),
      out_shape=jax.ShapeDtypeStruct((N, D), jnp.float32),
  )


_tc1 = _tc_layer(True)
_tc2 = _tc_layer(False)


def kernel(x, edge_index, W1l, b1l, W1r, W2l, b2l, W2r):
  src = edge_index[0].astype(jnp.int32).reshape(NW, NCHUNK, CHUNK)
  dst = edge_index[1].astype(jnp.int32).reshape(NW, NCHUNK, CHUNK)
  sum1, cnt = _agg_cnt(x, src, dst)
  h = _tc1(sum1, cnt, x, W1l, b1l.reshape(1, D), W1r)
  (sum2,) = _agg(h, src, dst)
  out = _tc2(sum2, cnt, h, W2l, b2l.reshape(1, D), W2r)
  return out


# speedup vs baseline: 1.3621x; 1.2811x over previous
import functools

import jax
import jax.numpy as jnp
from jax import lax
from jax.experimental import pallas as pl
from jax.experimental.pallas import tpu as pltpu
from jax.experimental.pallas import tpu_sc as plsc

N = 10000
E = 320000
D = 128

NC = 2
NS = 16
NW = NC * NS
EPW = E // NW
CHUNK = 40
NCHUNK = EPW // CHUNK
NB = 3
NPAD = 10240
RPS = NPAD // NS
ZROWS = 16
CW = 16


def _make_agg(with_counts):
  mesh = plsc.VectorSubcoreMesh(core_axis_name="c", subcore_axis_name="s")
  out_type = [jax.ShapeDtypeStruct((NC, NPAD, D), jnp.float32)]
  if with_counts:
    out_type.append(jax.ShapeDtypeStruct((NC, NPAD, CW), jnp.float32))
  scratch = [
      pltpu.VMEM((NCHUNK, CHUNK), jnp.int32),
      pltpu.VMEM((NCHUNK, CHUNK), jnp.int32),
      pltpu.VMEM((CHUNK, D), jnp.float32),
      pltpu.VMEM((CHUNK, D), jnp.float32),
      pltpu.VMEM((CHUNK, D), jnp.float32),
      pltpu.VMEM((ZROWS, D), jnp.float32),
      pltpu.VMEM_SHARED((NPAD, D), jnp.float32),
      pltpu.SemaphoreType.DMA,
      pltpu.SemaphoreType.DMA,
      pltpu.SemaphoreType.DMA,
  ]
  if with_counts:
    scratch += [
        pltpu.VMEM((ZROWS, CW), jnp.float32),
        pltpu.VMEM((CHUNK, CW), jnp.float32),
        pltpu.VMEM_SHARED((NPAD, CW), jnp.float32),
    ]

  @functools.partial(pl.kernel, mesh=mesh, out_type=out_type,
                     scratch_types=scratch,
                     compiler_params=pltpu.CompilerParams(
                         use_tc_tiling_on_sc=False))
  def agg(x_hbm, src_hbm, dst_hbm, *refs):
    if with_counts:
      out_hbm, cnt_hbm = refs[0], refs[1]
      (sidx, didx, rows0, rows1, rows2, zbuf, acc_sh,
       gs0, gs1, gs2, zcnt, ones, cnt_sh) = refs[2:]
    else:
      out_hbm = refs[0]
      (sidx, didx, rows0, rows1, rows2, zbuf, acc_sh,
       gs0, gs1, gs2) = refs[1:]
      cnt_hbm = zcnt = ones = cnt_sh = None

    c = lax.axis_index("c")
    s = lax.axis_index("s")
    wid = s * NC + c

    pltpu.sync_copy(src_hbm.at[wid], sidx)
    pltpu.sync_copy(dst_hbm.at[wid], didx)

    zv = jnp.zeros((16,), jnp.float32)
    ov = jnp.ones((16,), jnp.float32)

    def zb_body(i, carry):
      for j in range(D // 16):
        zbuf[i, pl.ds(j * 16, 16)] = zv
      return carry
    lax.fori_loop(0, ZROWS, zb_body, 0)

    if with_counts:
      def zc_body(i, carry):
        zcnt[i, pl.ds(0, 16)] = zv
        return carry
      lax.fori_loop(0, ZROWS, zc_body, 0)

      def on_body(i, carry):
        ones[i, pl.ds(0, 16)] = ov
        return carry
      lax.fori_loop(0, CHUNK, on_body, 0)

    for j in range(RPS // ZROWS):
      pltpu.sync_copy(zbuf, acc_sh.at[pl.ds(s * RPS + j * ZROWS, ZROWS)])
    if with_counts:
      for j in range(RPS // ZROWS):
        pltpu.sync_copy(zcnt, cnt_sh.at[pl.ds(s * RPS + j * ZROWS, ZROWS)])
    plsc.subcore_barrier()

    bufs = ((rows0, gs0), (rows1, gs1), (rows2, gs2))

    def wait_gather(rb, gsem):
      pltpu.make_async_copy(x_hbm.at[sidx.at[0]], rb, gsem).wait()

    def step(i, b, do_gather):
      rb, gsem = bufs[b]
      wait_gather(rb, gsem)
      pltpu.sync_copy(rb, acc_sh.at[didx.at[i]], add=True)
      if with_counts:
        pltpu.sync_copy(ones, cnt_sh.at[didx.at[i]], add=True)
      if do_gather:
        pltpu.async_copy(x_hbm.at[sidx.at[i + 3]], rb, gsem)

    pltpu.async_copy(x_hbm.at[sidx.at[0]], rows0, gs0)
    pltpu.async_copy(x_hbm.at[sidx.at[1]], rows1, gs1)
    pltpu.async_copy(x_hbm.at[sidx.at[2]], rows2, gs2)

    def steady(k, carry):
      i = 3 * k
      step(i, 0, True)
      step(i + 1, 1, True)
      step(i + 2, 2, True)
      return carry
    lax.fori_loop(0, NCHUNK // 3 - 1, steady, 0)

    step(246, 0, True)
    step(247, 1, False)
    step(248, 2, False)
    step(249, 0, False)

    plsc.subcore_barrier()

    pltpu.sync_copy(acc_sh.at[pl.ds(s * RPS, RPS)],
                    out_hbm.at[c, pl.ds(s * RPS, RPS)])
    if with_counts:
      pltpu.sync_copy(cnt_sh.at[pl.ds(s * RPS, RPS)],
                      cnt_hbm.at[c, pl.ds(s * RPS, RPS)])

  return agg


_agg_cnt = _make_agg(True)
_agg = _make_agg(False)

BLK = 1000


def _tc_body(sum_ref, cnt_ref, x_ref, wl_ref, bl_ref, wr_ref, o_ref, *, relu):
  s = sum_ref[0] + sum_ref[1]
  cnt = cnt_ref[...]
  c = cnt[0, :, 0:1] + cnt[1, :, 0:1]
  mean = s / jnp.maximum(c, 1.0)
  y = jnp.dot(mean, wl_ref[...], preferred_element_type=jnp.float32)
  y = y + jnp.dot(x_ref[...], wr_ref[...], preferred_element_type=jnp.float32)
  y = y + bl_ref[...]
  if relu:
    y = jnp.maximum(y, 0.0)
  o_ref[...] = y


def _tc_layer(relu):
  return pl.pallas_call(
      functools.partial(_tc_body, relu=relu),
      grid=(N // BLK,),
      in_specs=[
          pl.BlockSpec((NC, BLK, D), lambda i: (0, i, 0)),
          pl.BlockSpec((NC, BLK, CW), lambda i: (0, i, 0)),
          pl.BlockSpec((BLK, D), lambda i: (i, 0)),
          pl.BlockSpec((D, D), lambda i: (0, 0)),
          pl.BlockSpec((1, D), lambda i: (0, 0)),
          pl.BlockSpec((D, D), lambda i: (0, 0)),
      ],
      out_specs=pl.BlockSpec((BLK, D), lambda i: (i, 0)),
      out_shape=jax.ShapeDtypeStruct((N, D), jnp.float32),
  )


_tc1 = _tc_layer(True)
_tc2 = _tc_layer(False)


def kernel(x, edge_index, W1l, b1l, W1r, W2l, b2l, W2r):
  src = edge_index[0].astype(jnp.int32).reshape(NW, NCHUNK, CHUNK)
  dst = edge_index[1].astype(jnp.int32).reshape(NW, NCHUNK, CHUNK)
  sum1, cnt = _agg_cnt(x, src, dst)
  h = _tc1(sum1, cnt, x, W1l, b1l.reshape(1, D), W1r)
  (sum2,) = _agg(h, src, dst)
  out = _tc2(sum2, cnt, h, W2l, b2l.reshape(1, D), W2r)
  return out
